# Initial kernel scaffold; baseline (speedup 1.0000x reference)
import jax
import jax.numpy as jnp
from jax.experimental import pallas as pl


def kernel(hidden, state, W_hh, W_sh, W_ah, W_hs, W_r):
    raise NotImplementedError("write your pallas kernel here")


# speedup vs baseline: 1.2568x; 1.2568x over previous
import jax
import jax.numpy as jnp
from jax import lax
from jax.experimental import pallas as pl

_B = 8
_C = 256
_K = 32
_H = 512
_S = 128
_A = 16
_T = 8
_ITERS = 3

_DN0 = (((0,), (0,)), ((), ()))
_F32 = jnp.float32
_BF16 = jnp.bfloat16


def _dot(x, w):
    return jnp.dot(x, w, preferred_element_type=_F32)


def _planner_kernel(hid_ref, st_ref, whh_ref, wsh_ref, wah_ref, whs_ref,
                    wrh_ref, wrs_ref, eps_ref, out_ref):
    whh = whh_ref[...]
    wsh = wsh_ref[...]
    wah = wah_ref[...]
    whs = whs_ref[...]
    wrh = wrh_ref[...]
    wrs = wrs_ref[...]
    h0 = jnp.broadcast_to(hid_ref[0], (_C, _H))
    s0 = jnp.broadcast_to(st_ref[0], (_C, _S))

    row = lax.broadcasted_iota(jnp.int32, (_C, _C), 0)
    col = lax.broadcasted_iota(jnp.int32, (_C, _C), 1)
    ident = (row == col).astype(_F32)
    col_lt_row = col < row

    mean = [jnp.zeros((1, _A), _F32) for _ in range(_T)]
    std = [jnp.ones((1, _A), _F32) for _ in range(_T)]

    for it in range(_ITERS):
        h, s = h0, s0
        ret = jnp.zeros((_C, 1), _F32)
        acts = []
        for t in range(_T):
            eps_t = eps_ref[0, it * _T + t]
            a_t = mean[t] + std[t] * eps_t
            acts.append(a_t)
            h = jnp.tanh(_dot(h, whh) + _dot(s, wsh) + _dot(a_t, wah))
            s = jnp.tanh(_dot(h, whs))
            ret = ret + jnp.tanh(_dot(h, wrh) + _dot(s, wrs))

        rT = lax.dot_general(ret, ident, _DN0, preferred_element_type=_F32,
                             precision=lax.Precision.HIGHEST)
        beats = (rT > ret) | ((rT == ret) & col_lt_row)
        cnt = jnp.sum(beats.astype(_F32), axis=1, keepdims=True)
        m = (cnt < _K).astype(_F32)

        for t in range(_T):
            a_t = acts[t]
            sm = lax.dot_general(m, a_t, _DN0, preferred_element_type=_F32,
                                 precision=lax.Precision.HIGHEST) / _K
            cen = a_t - sm
            var = lax.dot_general(m, cen * cen, _DN0,
                                  preferred_element_type=_F32,
                                  precision=lax.Precision.HIGHEST) / _K
            mean[t] = sm
            std[t] = jnp.sqrt(var)

    out_ref[0] = mean[0]


def kernel(hidden, state, W_hh, W_sh, W_ah, W_hs, W_r):
    wrh = W_r[:_H]
    wrs = W_r[_H:]
    base_key = jax.random.key(42)
    eps = jnp.stack([
        jax.random.normal(jax.random.fold_in(base_key, it),
                          (_T, _B, _C, _A), dtype=jnp.float32)
        for it in range(_ITERS)
    ])
    eps = eps.transpose(2, 0, 1, 3, 4).reshape(_B, _ITERS * _T, _C, _A)

    out = pl.pallas_call(
        _planner_kernel,
        grid=(_B,),
        in_specs=[
            pl.BlockSpec((1, 1, _H), lambda b: (b, 0, 0)),
            pl.BlockSpec((1, 1, _S), lambda b: (b, 0, 0)),
            pl.BlockSpec((_H, _H), lambda b: (0, 0)),
            pl.BlockSpec((_S, _H), lambda b: (0, 0)),
            pl.BlockSpec((_A, _H), lambda b: (0, 0)),
            pl.BlockSpec((_H, _S), lambda b: (0, 0)),
            pl.BlockSpec((_H, 1), lambda b: (0, 0)),
            pl.BlockSpec((_S, 1), lambda b: (0, 0)),
            pl.BlockSpec((1, _ITERS * _T, _C, _A), lambda b: (b, 0, 0, 0)),
        ],
        out_specs=pl.BlockSpec((1, 1, _A), lambda b: (b, 0, 0)),
        out_shape=jax.ShapeDtypeStruct((_B, 1, _A), jnp.float32),
    )(hidden[:, None, :], state[:, None, :], W_hh, W_sh, W_ah, W_hs,
      wrh, wrs, eps)
    return out[:, 0, :]


# speedup vs baseline: 1.5702x; 1.2494x over previous
import functools

import jax
import jax.numpy as jnp
import numpy as np
from jax import lax
from jax.experimental import pallas as pl

_B = 8
_C = 256
_K = 32
_H = 512
_S = 128
_A = 16
_T = 8
_ITERS = 3

_F32 = jnp.float32
_BF16 = jnp.bfloat16


def _dot(x, w):
    return jnp.dot(x, w, preferred_element_type=_F32)


def _planner_kernel(hid_ref, st_ref, whh_ref, wsh_ref, wah_ref, whs_ref,
                    wrh_ref, wrs_ref, eps_ref, out_ref):
    whh = whh_ref[...]
    wsh = wsh_ref[...]
    wah = wah_ref[...]
    whs = whs_ref[...]
    wrh = wrh_ref[...]
    wrs = wrs_ref[...]
    h0 = jnp.broadcast_to(hid_ref[0], (_C, _H))
    s0 = jnp.broadcast_to(st_ref[0], (_C, _S))

    row = lax.broadcasted_iota(jnp.int32, (_C, _C), 0)
    col = lax.broadcasted_iota(jnp.int32, (_C, _C), 1)
    col_lt_row = col < row

    mean = [jnp.zeros((1, _A), _F32) for _ in range(_T)]
    std = [jnp.ones((1, _A), _F32) for _ in range(_T)]

    for it in range(_ITERS):
        h, s = h0, s0
        ret = jnp.zeros((_C, 1), _F32)
        acts = []
        for t in range(_T):
            eps_t = eps_ref[0, it * _T + t]
            a_t = mean[t] + std[t] * eps_t
            acts.append(a_t)
            h = jnp.tanh(_dot(h, whh) + _dot(s, wsh) + _dot(a_t, wah))
            s = jnp.tanh(_dot(h, whs))
            ret = ret + jnp.tanh(_dot(h, wrh) + _dot(s, wrs))

        rT = jnp.transpose(ret)
        beats = (rT > ret) | ((rT == ret) & col_lt_row)
        cnt = jnp.sum(beats.astype(_F32), axis=1, keepdims=True)
        m = (cnt < _K).astype(_F32)

        for t in range(_T):
            a_t = acts[t]
            sm = jnp.sum(a_t * m, axis=0, keepdims=True) / _K
            cen = a_t - sm
            var = jnp.sum(cen * cen * m, axis=0, keepdims=True) / _K
            mean[t] = sm
            std[t] = jnp.sqrt(var)

    out_ref[0] = mean[0]


def kernel(hidden, state, W_hh, W_sh, W_ah, W_hs, W_r):
    wrh = W_r[:_H]
    wrs = W_r[_H:]
    base_key = jax.random.key(42)
    eps = jnp.stack([
        jax.random.normal(jax.random.fold_in(base_key, it),
                          (_T, _B, _C, _A), dtype=jnp.float32)
        for it in range(_ITERS)
    ])
    eps = eps.transpose(2, 0, 1, 3, 4).reshape(_B, _ITERS * _T, _C, _A)

    out = pl.pallas_call(
        _planner_kernel,
        grid=(_B,),
        in_specs=[
            pl.BlockSpec((1, 1, _H), lambda b: (b, 0, 0)),
            pl.BlockSpec((1, 1, _S), lambda b: (b, 0, 0)),
            pl.BlockSpec((_H, _H), lambda b: (0, 0)),
            pl.BlockSpec((_S, _H), lambda b: (0, 0)),
            pl.BlockSpec((_A, _H), lambda b: (0, 0)),
            pl.BlockSpec((_H, _S), lambda b: (0, 0)),
            pl.BlockSpec((_H, 1), lambda b: (0, 0)),
            pl.BlockSpec((_S, 1), lambda b: (0, 0)),
            pl.BlockSpec((1, _ITERS * _T, _C, _A), lambda b: (b, 0, 0, 0)),
        ],
        out_specs=pl.BlockSpec((1, 1, _A), lambda b: (b, 0, 0)),
        out_shape=jax.ShapeDtypeStruct((_B, 1, _A), jnp.float32),
    )(hidden[:, None, :], state[:, None, :], W_hh, W_sh, W_ah, W_hs,
      wrh, wrs, eps)
    return out[:, 0, :]


# speedup vs baseline: 1.5730x; 1.0018x over previous
import jax
import jax.numpy as jnp
from jax import lax
from jax.experimental import pallas as pl

_B = 8
_C = 256
_K = 32
_H = 512
_S = 128
_A = 16
_T = 8
_ITERS = 3

_F32 = jnp.float32


def _dot(x, w):
    return jnp.dot(x, w, preferred_element_type=_F32)


def _planner_kernel(hid_ref, st_ref, whh_ref, wsh_ref, wah_ref, whs_ref,
                    wrh_ref, wrs_ref, eps_ref, out_ref):
    whh = whh_ref[...]
    wsh = wsh_ref[...]
    wah = wah_ref[...]
    whs = whs_ref[...]
    wrh = wrh_ref[...]
    wrs = wrs_ref[...]
    h0 = jnp.broadcast_to(hid_ref[0], (_C, _H))
    s0 = jnp.broadcast_to(st_ref[0], (_C, _S))

    row = lax.broadcasted_iota(jnp.int32, (_C, _C), 0)
    col = lax.broadcasted_iota(jnp.int32, (_C, _C), 1)
    col_lt_row = col < row

    mean = [jnp.zeros((1, _A), _F32) for _ in range(_T)]
    std = [jnp.ones((1, _A), _F32) for _ in range(_T)]

    for it in range(_ITERS):
        h, s = h0, s0
        ret = jnp.zeros((_C, 1), _F32)
        acts = []
        for t in range(_T):
            eps_t = eps_ref[it * _T + t, 0]
            a_t = mean[t] + std[t] * eps_t
            acts.append(a_t)
            h = jnp.tanh(_dot(h, whh) + _dot(s, wsh) + _dot(a_t, wah))
            s = jnp.tanh(_dot(h, whs))
            ret = ret + jnp.tanh(_dot(h, wrh) + _dot(s, wrs))

        rT = jnp.transpose(ret)
        beats = (rT > ret) | ((rT == ret) & col_lt_row)
        cnt = jnp.sum(beats.astype(_F32), axis=1, keepdims=True)
        m = (cnt < _K).astype(_F32)

        for t in range(_T):
            a_t = acts[t]
            sm = jnp.sum(a_t * m, axis=0, keepdims=True) / _K
            cen = a_t - sm
            var = jnp.sum(cen * cen * m, axis=0, keepdims=True) / _K
            mean[t] = sm
            std[t] = jnp.sqrt(var)

    out_ref[0] = mean[0]


def kernel(hidden, state, W_hh, W_sh, W_ah, W_hs, W_r):
    wrh = W_r[:_H]
    wrs = W_r[_H:]
    base_key = jax.random.key(42)
    eps = jnp.stack([
        jax.random.normal(jax.random.fold_in(base_key, it),
                          (_T, _B, _C, _A), dtype=jnp.float32)
        for it in range(_ITERS)
    ])
    eps = eps.reshape(_ITERS * _T, _B, _C, _A)

    out = pl.pallas_call(
        _planner_kernel,
        grid=(_B,),
        in_specs=[
            pl.BlockSpec((1, 1, _H), lambda b: (b, 0, 0)),
            pl.BlockSpec((1, 1, _S), lambda b: (b, 0, 0)),
            pl.BlockSpec((_H, _H), lambda b: (0, 0)),
            pl.BlockSpec((_S, _H), lambda b: (0, 0)),
            pl.BlockSpec((_A, _H), lambda b: (0, 0)),
            pl.BlockSpec((_H, _S), lambda b: (0, 0)),
            pl.BlockSpec((_H, 1), lambda b: (0, 0)),
            pl.BlockSpec((_S, 1), lambda b: (0, 0)),
            pl.BlockSpec((_ITERS * _T, 1, _C, _A), lambda b: (0, b, 0, 0)),
        ],
        out_specs=pl.BlockSpec((1, 1, _A), lambda b: (b, 0, 0)),
        out_shape=jax.ShapeDtypeStruct((_B, 1, _A), jnp.float32),
    )(hidden[:, None, :], state[:, None, :], W_hh, W_sh, W_ah, W_hs,
      wrh, wrs, eps)
    return out[:, 0, :]


# speedup vs baseline: 1.6590x; 1.0547x over previous
import jax
import jax.numpy as jnp
from jax import lax
from jax.experimental import pallas as pl

_B = 8
_C = 256
_K = 32
_H = 512
_S = 128
_A = 16
_T = 8
_ITERS = 3

_F32 = jnp.float32


def _dot(x, w):
    return jnp.dot(x, w, preferred_element_type=_F32)


_G = 2


def _planner_kernel(hid_ref, st_ref, whh_ref, wsh_ref, wah_ref, whs_ref,
                    wrh_ref, wrs_ref, eps_ref, out_ref):
    whh = whh_ref[...]
    wsh = wsh_ref[...]
    wah = wah_ref[...]
    whs = whs_ref[...]
    wrh = wrh_ref[...]
    wrs = wrs_ref[...]
    h0 = jnp.concatenate(
        [jnp.broadcast_to(hid_ref[j, 0], (_C, _H)) for j in range(_G)], axis=0)
    s0 = jnp.concatenate(
        [jnp.broadcast_to(st_ref[j, 0], (_C, _S)) for j in range(_G)], axis=0)

    row = lax.broadcasted_iota(jnp.int32, (_C, _C), 0)
    col = lax.broadcasted_iota(jnp.int32, (_C, _C), 1)
    col_lt_row = col < row

    mean = [[jnp.zeros((1, _A), _F32) for _ in range(_T)] for _ in range(_G)]
    std = [[jnp.ones((1, _A), _F32) for _ in range(_T)] for _ in range(_G)]

    for it in range(_ITERS):
        h, s = h0, s0
        ret = jnp.zeros((_G * _C, 1), _F32)
        acts = []
        for t in range(_T):
            a_t = jnp.concatenate(
                [mean[j][t] + std[j][t] * eps_ref[it * _T + t, j]
                 for j in range(_G)], axis=0)
            acts.append(a_t)
            h = jnp.tanh(_dot(h, whh) + _dot(s, wsh) + _dot(a_t, wah))
            s = jnp.tanh(_dot(h, whs))
            ret = ret + jnp.tanh(_dot(h, wrh) + _dot(s, wrs))

        ms = []
        for j in range(_G):
            rb = ret[j * _C:(j + 1) * _C, :]
            rT = jnp.transpose(rb)
            beats = (rT > rb) | ((rT == rb) & col_lt_row)
            cnt = jnp.sum(beats.astype(_F32), axis=1, keepdims=True)
            ms.append((cnt < _K).astype(_F32))

        for t in range(_T):
            for j in range(_G):
                a_t = acts[t][j * _C:(j + 1) * _C, :]
                sm = jnp.sum(a_t * ms[j], axis=0, keepdims=True) / _K
                cen = a_t - sm
                var = jnp.sum(cen * cen * ms[j], axis=0, keepdims=True) / _K
                mean[j][t] = sm
                std[j][t] = jnp.sqrt(var)

    for j in range(_G):
        out_ref[j] = mean[j][0]


def kernel(hidden, state, W_hh, W_sh, W_ah, W_hs, W_r):
    wrh = W_r[:_H]
    wrs = W_r[_H:]
    base_key = jax.random.key(42)
    eps = jnp.stack([
        jax.random.normal(jax.random.fold_in(base_key, it),
                          (_T, _B, _C, _A), dtype=jnp.float32)
        for it in range(_ITERS)
    ])
    eps = eps.reshape(_ITERS * _T, _B, _C, _A)

    out = pl.pallas_call(
        _planner_kernel,
        grid=(_B // _G,),
        in_specs=[
            pl.BlockSpec((_G, 1, _H), lambda b: (b, 0, 0)),
            pl.BlockSpec((_G, 1, _S), lambda b: (b, 0, 0)),
            pl.BlockSpec((_H, _H), lambda b: (0, 0)),
            pl.BlockSpec((_S, _H), lambda b: (0, 0)),
            pl.BlockSpec((_A, _H), lambda b: (0, 0)),
            pl.BlockSpec((_H, _S), lambda b: (0, 0)),
            pl.BlockSpec((_H, 1), lambda b: (0, 0)),
            pl.BlockSpec((_S, 1), lambda b: (0, 0)),
            pl.BlockSpec((_ITERS * _T, _G, _C, _A), lambda b: (0, b, 0, 0)),
        ],
        out_specs=pl.BlockSpec((_G, 1, _A), lambda b: (b, 0, 0)),
        out_shape=jax.ShapeDtypeStruct((_B, 1, _A), jnp.float32),
    )(hidden[:, None, :], state[:, None, :], W_hh, W_sh, W_ah, W_hs,
      wrh, wrs, eps)
    return out[:, 0, :]
